# baseline (speedup 1.0000x reference)
import functools

import jax
import jax.numpy as jnp
from jax import lax
from jax.experimental import pallas as pl
from jax.experimental.pallas import tpu as pltpu
from jax.experimental.pallas import tpu_sc as plsc

NBINS = 256
NCH = 3
KEYS = NBINS * NCH
B = 64
HW = 512 * 512
PER_IMG = HW * NCH
L = 16
NWORK = 32
IMGS_PER_W = B // NWORK
CHUNK = 49152
NCHUNKS = PER_IMG // CHUNK


def _hist_body(x_hbm, out_hbm, buf0, buf1, hist, histf, sem0, sem1):
    wid = lax.axis_index("s") * 2 + lax.axis_index("c")
    lane = lax.iota(jnp.int32, L)
    ones = jnp.ones((L,), jnp.int32)
    consts = [lax.rem(lane + (p * L) % 3, jnp.int32(3)) + lane * KEYS
              for p in range(3)]
    bufs = (buf0, buf1)
    sems = (sem0, sem1)

    for img in range(IMGS_PER_W):
        bi = wid * IMGS_PER_W + img
        base = bi * PER_IMG

        @pl.loop(0, L * KEYS, step=L)
        def _zero(j):
            hist[pl.ds(j, L)] = jnp.zeros((L,), jnp.int32)

        descs = [None, None]
        descs[0] = pltpu.async_copy(
            x_hbm.at[pl.ds(base, CHUNK)], bufs[0], sems[0])
        for c in range(NCHUNKS):
            par = c % 2
            if c + 1 < NCHUNKS:
                nxt = (c + 1) % 2
                descs[nxt] = pltpu.async_copy(
                    x_hbm.at[pl.ds(base + (c + 1) * CHUNK, CHUNK)],
                    bufs[nxt], sems[nxt])
            descs[par].wait()
            buf = bufs[par]

            @pl.loop(0, CHUNK, step=48, unroll=4)
            def _accum(o):
                for p in range(3):
                    v = buf[pl.ds(o + p * L, L)]
                    b = (v * jnp.float32(NBINS)).astype(jnp.int32)
                    b = jnp.minimum(jnp.maximum(b, 0), NBINS - 1)
                    plsc.addupdate_scatter(hist, [b * 3 + consts[p]], ones)

        @pl.loop(0, KEYS, step=L)
        def _reduce(j):
            acc = hist[pl.ds(j, L)]
            for l in range(1, L):
                acc = acc + hist[pl.ds(l * KEYS + j, L)]
            histf[pl.ds(j, L)] = acc.astype(jnp.float32) * jnp.float32(
                1.0 / HW)
        pltpu.sync_copy(histf, out_hbm.at[bi])


@jax.jit
def _hist_sc(x_flat):
    mesh = plsc.VectorSubcoreMesh(core_axis_name="c", subcore_axis_name="s")
    f = pl.kernel(
        _hist_body,
        out_type=jax.ShapeDtypeStruct((B, KEYS), jnp.float32),
        mesh=mesh,
        compiler_params=pltpu.CompilerParams(needs_layout_passes=False),
        scratch_types=[
            pltpu.VMEM((CHUNK,), jnp.float32),
            pltpu.VMEM((CHUNK,), jnp.float32),
            pltpu.VMEM((L * KEYS,), jnp.int32),
            pltpu.VMEM((KEYS,), jnp.float32),
            pltpu.SemaphoreType.DMA,
            pltpu.SemaphoreType.DMA,
        ],
    )
    return f(x_flat)


def kernel(inputs):
    x = inputs.reshape(-1)
    out = _hist_sc(x)
    return out.reshape(B, NBINS, NCH)


# speedup vs baseline: 18.5957x; 18.5957x over previous
import jax
import jax.numpy as jnp
from jax import lax
from jax.experimental import pallas as pl
from jax.experimental.pallas import tpu as pltpu
from jax.experimental.pallas import tpu_sc as plsc

NBINS = 256
NCH = 3
KEYS = NBINS * NCH
B = 64
HW = 512 * 512
L = 16
NWORK = 32
IMGS_PER_W = B // NWORK
ROWS_PER_IMG = NCH * 512
CROWS = 64
NCHUNKS = ROWS_PER_IMG // CROWS


def _hist_body(x_hbm, out_hbm, buf0, buf1, hist, histf, sem0, sem1):
    wid = lax.axis_index("s") * 2 + lax.axis_index("c")
    lane = lax.iota(jnp.int32, L)
    ones = jnp.ones((L,), jnp.int32)
    lane_off = lane * KEYS
    bufs = (buf0, buf1)
    sems = (sem0, sem1)

    for img in range(IMGS_PER_W):
        bi = wid * IMGS_PER_W + img
        row_base = bi * ROWS_PER_IMG

        @pl.loop(0, L * KEYS, step=L)
        def _zero(j):
            hist[pl.ds(j, L)] = jnp.zeros((L,), jnp.int32)

        descs = [None, None]
        descs[0] = pltpu.async_copy(
            x_hbm.at[pl.ds(row_base, CROWS), :], bufs[0], sems[0])
        for k in range(NCHUNKS):
            par = k % 2
            if k + 1 < NCHUNKS:
                nxt = (k + 1) % 2
                descs[nxt] = pltpu.async_copy(
                    x_hbm.at[pl.ds(row_base + (k + 1) * CROWS, CROWS), :],
                    bufs[nxt], sems[nxt])
            descs[par].wait()
            buf = bufs[par]
            cvec = lane_off + (k // 8) * NBINS

            @pl.loop(0, CROWS * 512, step=L, unroll=8)
            def _accum(o):
                r = lax.shift_right_logical(o, 9)
                c = lax.bitwise_and(o, 511)
                v = buf[r, pl.ds(c, L)]
                b = (v * jnp.float32(NBINS)).astype(jnp.int32)
                b = jnp.minimum(jnp.maximum(b, 0), NBINS - 1)
                plsc.addupdate_scatter(hist, [b + cvec], ones)

        @pl.loop(0, KEYS, step=L)
        def _reduce(j):
            acc = hist[pl.ds(j, L)]
            for l in range(1, L):
                acc = acc + hist[pl.ds(l * KEYS + j, L)]
            histf[pl.ds(j, L)] = acc.astype(jnp.float32) * jnp.float32(
                1.0 / HW)
        pltpu.sync_copy(histf, out_hbm.at[pl.ds(bi * KEYS, KEYS)])


@jax.jit
def _hist_sc(x2):
    mesh = plsc.VectorSubcoreMesh(core_axis_name="c", subcore_axis_name="s")
    f = pl.kernel(
        _hist_body,
        out_type=jax.ShapeDtypeStruct((B * KEYS,), jnp.float32),
        mesh=mesh,
        compiler_params=pltpu.CompilerParams(
            needs_layout_passes=False, use_tc_tiling_on_sc=True),
        scratch_types=[
            pltpu.VMEM((CROWS, 512), jnp.float32),
            pltpu.VMEM((CROWS, 512), jnp.float32),
            pltpu.VMEM((L * KEYS,), jnp.int32),
            pltpu.VMEM((KEYS,), jnp.float32),
            pltpu.SemaphoreType.DMA,
            pltpu.SemaphoreType.DMA,
        ],
    )
    return f(x2)


def kernel(inputs):
    x2 = lax.transpose(inputs, (0, 3, 1, 2)).reshape(B * ROWS_PER_IMG, 512)
    out = _hist_sc(x2)
    return out.reshape(B, NCH, NBINS).transpose(0, 2, 1)
